# Initial kernel scaffold; baseline (speedup 1.0000x reference)
import jax
import jax.numpy as jnp
from jax.experimental import pallas as pl


def kernel(x, edge_index, W_node, b_node, W0, b0, g0, t0, W1, b1, g1, t1, W2, b2, g2, t2, W_cls, b_cls):
    raise NotImplementedError("write your pallas kernel here")


# speedup vs baseline: 8.6597x; 8.6597x over previous
import functools

import jax
import jax.numpy as jnp
from jax import lax
from jax.experimental import pallas as pl
from jax.experimental.pallas import tpu as pltpu
from jax.experimental.pallas import tpu_sc as plsc

N = 10000
MPAD = 10240
E = 160000
EPAD = 163840
NPADROWS = MPAD - N
DH = 512
NCHUNK = 4
BM = 256
GRID_M = MPAD // BM
TILE_ROWS = MPAD // 16


@functools.cache
def _sc_kernels():
    mesh = plsc.VectorSubcoreMesh(core_axis_name="c", subcore_axis_name="s",
                                  num_cores=2, num_subcores=16)

    @functools.partial(
        pl.kernel,
        out_type=jax.ShapeDtypeStruct((2, MPAD), jnp.float32),
        mesh=mesh,
        scratch_types=[
            pltpu.VMEM((40, 128), jnp.int32),
            pltpu.VMEM((128,), jnp.float32),
            pltpu.VMEM_SHARED((MPAD,), jnp.float32),
        ],
    )
    def _sc_degree(dst_idx, zeros1d, ones1d, deg_out, idx_v, ones_v, deg_sh):
        c = lax.axis_index("c")
        s = lax.axis_index("s")
        pltpu.sync_copy(zeros1d.at[pl.ds(0, TILE_ROWS)],
                        deg_sh.at[pl.ds(s * TILE_ROWS, TILE_ROWS)])
        pltpu.sync_copy(ones1d, ones_v)
        pltpu.sync_copy(dst_idx.at[c * 16 + s], idx_v)
        plsc.subcore_barrier()

        def body(j, carry):
            pltpu.sync_copy(ones_v, deg_sh.at[idx_v.at[j]], add=True)
            return carry

        lax.fori_loop(0, 40, body, 0)
        plsc.subcore_barrier()
        pltpu.sync_copy(deg_sh.at[pl.ds(s * TILE_ROWS, TILE_ROWS)],
                        deg_out.at[c, pl.ds(s * TILE_ROWS, TILE_ROWS)])

    @functools.partial(
        pl.kernel,
        out_type=jax.ShapeDtypeStruct((NCHUNK * MPAD, 128), jnp.float32),
        mesh=mesh,
        scratch_types=[
            pltpu.VMEM((80, 128), jnp.int32),
            pltpu.VMEM((80, 128), jnp.int32),
            pltpu.VMEM((128, 128), jnp.float32),
            pltpu.VMEM_SHARED((MPAD, 128), jnp.float32),
            pltpu.SemaphoreType.DMA,
        ],
    )
    def _sc_msgpass(p_rows, src_idx, dst_idx, zeros2d, s_out,
                    idx_s, idx_d, rows_v, agg_sh, gsem):
        c = lax.axis_index("c")
        s = lax.axis_index("s")
        pltpu.sync_copy(dst_idx.at[s], idx_d)
        for ci in range(2):
            chunk = c * 2 + ci
            pltpu.sync_copy(zeros2d,
                            agg_sh.at[pl.ds(s * TILE_ROWS, TILE_ROWS)])
            pltpu.sync_copy(src_idx.at[chunk * 16 + s], idx_s)
            plsc.subcore_barrier()

            def body(j, carry):
                pltpu.async_copy(p_rows.at[idx_s.at[j]], rows_v, gsem).wait()
                pltpu.sync_copy(rows_v, agg_sh.at[idx_d.at[j]], add=True)
                return carry

            lax.fori_loop(0, 80, body, 0)
            plsc.subcore_barrier()
            pltpu.sync_copy(
                agg_sh.at[pl.ds(s * TILE_ROWS, TILE_ROWS)],
                s_out.at[pl.ds(chunk * MPAD + s * TILE_ROWS, TILE_ROWS)])
            plsc.subcore_barrier()

    return _sc_degree, _sc_msgpass


def _tc_enc_body(x_ref, degp_ref, wn_ref, bn_ref, w0_ref, p_ref, dinv_ref):
    m = pl.program_id(0)
    deg = degp_ref[:, 0:1] + degp_ref[:, 1:2] + 1.0
    rows = jax.lax.broadcasted_iota(jnp.int32, (BM, 1), 0) + m * BM
    dinv = jnp.where(rows < N, lax.rsqrt(deg), 0.0)
    dinv_ref[...] = dinv
    h = jnp.maximum(
        jnp.dot(x_ref[...], wn_ref[...], preferred_element_type=jnp.float32)
        + bn_ref[...], 0.0)
    p = jnp.dot(h, w0_ref[...], preferred_element_type=jnp.float32) * dinv
    for cc in range(NCHUNK):
        p_ref[cc] = p[:, cc * 128:(cc + 1) * 128]


def _tc_combine_body(s_ref, p_ref, dinv_ref, b_ref, z_ref, stat_ref):
    m = pl.program_id(0)

    @pl.when(m == 0)
    def _():
        stat_ref[...] = jnp.zeros_like(stat_ref)

    dinv = dinv_ref[...]
    rows = jax.lax.broadcasted_iota(jnp.int32, (BM, 1), 0) + m * BM
    mask = rows < N
    for cc in range(NCHUNK):
        z = (s_ref[cc] + p_ref[cc]) * dinv + b_ref[:, cc * 128:(cc + 1) * 128]
        z_ref[:, cc * 128:(cc + 1) * 128] = z
        zm = jnp.where(mask, z, 0.0)
        stat_ref[0:1, cc * 128:(cc + 1) * 128] += jnp.sum(zm, axis=0,
                                                          keepdims=True)
        stat_ref[1:2, cc * 128:(cc + 1) * 128] += jnp.sum(zm * zm, axis=0,
                                                          keepdims=True)


def _bn_coeffs(stat, g, t):
    mu = stat[0:1, :] * (1.0 / N)
    var = stat[1:2, :] * (1.0 / N) - mu * mu
    a = lax.rsqrt(var + 1e-5) * g
    return a, t - mu * a


def _tc_normmm_body(z_ref, stat_ref, g_ref, t_ref, dinv_ref, w_ref, p_ref):
    a, c0 = _bn_coeffs(stat_ref[...], g_ref[...], t_ref[...])
    h = jnp.maximum(z_ref[...] * a + c0, 0.0)
    p = jnp.dot(h, w_ref[...], preferred_element_type=jnp.float32) \
        * dinv_ref[...]
    for cc in range(NCHUNK):
        p_ref[cc] = p[:, cc * 128:(cc + 1) * 128]


def _tc_final_body(z_ref, stat_ref, g_ref, t_ref, w_ref, b_ref, o_ref):
    a, c0 = _bn_coeffs(stat_ref[...], g_ref[...], t_ref[...])
    h = jnp.maximum(z_ref[...] * a + c0, 0.0)
    o_ref[...] = jnp.dot(h, w_ref[...],
                         preferred_element_type=jnp.float32) + b_ref[...]


def _row_spec(width):
    return pl.BlockSpec((BM, width), lambda m: (m, 0))


def _full_spec(shape):
    return pl.BlockSpec(shape, lambda m: tuple(0 for _ in shape))


_CHUNK_SPEC = pl.BlockSpec((NCHUNK, BM, 128), lambda m: (0, m, 0))
_P_SHAPE = jax.ShapeDtypeStruct((NCHUNK, MPAD, 128), jnp.float32)


def _tc_enc(x_pad, degp, w_node, b_node, w0):
    return pl.pallas_call(
        _tc_enc_body,
        grid=(GRID_M,),
        in_specs=[_row_spec(256), _row_spec(2), _full_spec((256, DH)),
                  _full_spec((1, DH)), _full_spec((DH, DH))],
        out_specs=[_CHUNK_SPEC, _row_spec(1)],
        out_shape=[_P_SHAPE, jax.ShapeDtypeStruct((MPAD, 1), jnp.float32)],
    )(x_pad, degp, w_node, b_node, w0)


def _tc_combine(s_agg, p, dinv, b):
    return pl.pallas_call(
        _tc_combine_body,
        grid=(GRID_M,),
        in_specs=[_CHUNK_SPEC, _CHUNK_SPEC, _row_spec(1), _full_spec((1, DH))],
        out_specs=[_row_spec(DH), _full_spec((8, DH))],
        out_shape=[jax.ShapeDtypeStruct((MPAD, DH), jnp.float32),
                   jax.ShapeDtypeStruct((8, DH), jnp.float32)],
    )(s_agg, p, dinv, b)


def _tc_normmm(z, stat, g, t, dinv, w):
    return pl.pallas_call(
        _tc_normmm_body,
        grid=(GRID_M,),
        in_specs=[_row_spec(DH), _full_spec((8, DH)), _full_spec((1, DH)),
                  _full_spec((1, DH)), _row_spec(1), _full_spec((DH, DH))],
        out_specs=_CHUNK_SPEC,
        out_shape=_P_SHAPE,
    )(z, stat, g, t, dinv, w)


def _tc_final(z, stat, g, t, w_cls, b_cls):
    return pl.pallas_call(
        _tc_final_body,
        grid=(GRID_M,),
        in_specs=[_row_spec(DH), _full_spec((8, DH)), _full_spec((1, DH)),
                  _full_spec((1, DH)), _full_spec((DH, 128)),
                  _full_spec((1, 128))],
        out_specs=_row_spec(128),
        out_shape=jax.ShapeDtypeStruct((MPAD, 128), jnp.float32),
    )(z, stat, g, t, w_cls, b_cls)


def kernel(x, edge_index, W_node, b_node, W0, b0, g0, t0, W1, b1, g1, t1,
           W2, b2, g2, t2, W_cls, b_cls):
    f32 = jnp.float32
    pad = (N + (jnp.arange(EPAD - E, dtype=jnp.int32) % NPADROWS))
    src = jnp.concatenate([edge_index[0], pad])
    dst = jnp.concatenate([edge_index[1], pad])
    dst_deg_idx = dst.reshape(32, 40, 128)
    chunk_off = (jnp.arange(NCHUNK, dtype=jnp.int32) * MPAD)[:, None]
    src_mp_idx = (src[None, :] + chunk_off).reshape(NCHUNK * 16, 80, 128)
    dst_mp_idx = dst.reshape(16, 80, 128)

    zeros1d = jnp.zeros((MPAD,), f32)
    ones1d = jnp.ones((128,), f32)
    zeros2d = jnp.zeros((TILE_ROWS, 128), f32)

    sc_degree, sc_msgpass = _sc_kernels()
    degp = sc_degree(dst_deg_idx, zeros1d, ones1d)
    degp_t = degp.T

    x_pad = jnp.pad(x, ((0, MPAD - N), (0, 0)))
    b_node2 = b_node.reshape(1, DH)

    p, dinv = _tc_enc(x_pad, degp_t, W_node, b_node2, W0)

    zs = None
    for (w_next, b_l, g_l, t_l) in ((W1, b0, g0, t0), (W2, b1, g1, t1),
                                    (None, b2, g2, t2)):
        s_flat = sc_msgpass(p.reshape(NCHUNK * MPAD, 128), src_mp_idx,
                            dst_mp_idx, zeros2d)
        s_agg = s_flat.reshape(NCHUNK, MPAD, 128)
        z, stat = _tc_combine(s_agg, p, dinv, b_l.reshape(1, DH))
        if w_next is not None:
            p = _tc_normmm(z, stat, g_l.reshape(1, DH), t_l.reshape(1, DH),
                           dinv, w_next)
        else:
            zs = (z, stat, g_l, t_l)

    z, stat, g_l, t_l = zs
    out = _tc_final(z, stat, g_l.reshape(1, DH), t_l.reshape(1, DH),
                    W_cls, b_cls.reshape(1, 128))
    return out[:N]


# speedup vs baseline: 11.4228x; 1.3191x over previous
import functools

import jax
import jax.numpy as jnp
from jax import lax
from jax.experimental import pallas as pl
from jax.experimental.pallas import tpu as pltpu
from jax.experimental.pallas import tpu_sc as plsc

N = 10000
MPAD = 10240
E = 160000
EPAD = 163840
NPADROWS = MPAD - N
DH = 512
NCHUNK = 4
BM = 256
GRID_M = MPAD // BM
TILE_ROWS = MPAD // 16


@functools.cache
def _sc_kernels():
    mesh = plsc.VectorSubcoreMesh(core_axis_name="c", subcore_axis_name="s",
                                  num_cores=2, num_subcores=16)

    @functools.partial(
        pl.kernel,
        out_type=jax.ShapeDtypeStruct((2, MPAD), jnp.float32),
        mesh=mesh,
        scratch_types=[
            pltpu.VMEM((40, 128), jnp.int32),
            pltpu.VMEM((128,), jnp.float32),
            pltpu.VMEM_SHARED((MPAD,), jnp.float32),
        ],
    )
    def _sc_degree(dst_idx, zeros1d, ones1d, deg_out, idx_v, ones_v, deg_sh):
        c = lax.axis_index("c")
        s = lax.axis_index("s")
        pltpu.sync_copy(zeros1d.at[pl.ds(0, TILE_ROWS)],
                        deg_sh.at[pl.ds(s * TILE_ROWS, TILE_ROWS)])
        pltpu.sync_copy(ones1d, ones_v)
        pltpu.sync_copy(dst_idx.at[c * 16 + s], idx_v)
        plsc.subcore_barrier()

        def body(j, carry):
            pltpu.sync_copy(ones_v, deg_sh.at[idx_v.at[j]], add=True)
            return carry

        lax.fori_loop(0, 40, body, 0)
        plsc.subcore_barrier()
        pltpu.sync_copy(deg_sh.at[pl.ds(s * TILE_ROWS, TILE_ROWS)],
                        deg_out.at[c, pl.ds(s * TILE_ROWS, TILE_ROWS)])

    @functools.partial(
        pl.kernel,
        out_type=jax.ShapeDtypeStruct((NCHUNK * MPAD, 128), jnp.float32),
        mesh=mesh,
        scratch_types=[
            pltpu.VMEM((40, 128), jnp.int32),
            pltpu.VMEM((40, 128), jnp.int32),
            pltpu.VMEM((128, 128), jnp.float32),
            pltpu.VMEM((128, 128), jnp.float32),
            pltpu.VMEM_SHARED((MPAD, 128), jnp.float32),
            pltpu.SemaphoreType.DMA,
            pltpu.SemaphoreType.DMA,
        ],
    )
    def _sc_msgpass(p_rows, src_idx, dst_idx, zeros2d, s_out,
                    idx_s, idx_d, rows0, rows1, agg_sh, sem0, sem1):
        c = lax.axis_index("c")
        s = lax.axis_index("s")
        for ci in range(2):
            chunk = c * 2 + ci
            pltpu.sync_copy(zeros2d,
                            agg_sh.at[pl.ds(s * TILE_ROWS, TILE_ROWS)])
            plsc.subcore_barrier()
            for half in range(2):
                pltpu.sync_copy(src_idx.at[(chunk * 16 + s) * 2 + half],
                                idx_s)
                pltpu.sync_copy(dst_idx.at[s * 2 + half], idx_d)

                bufs = (rows0, rows1)
                sems = (sem0, sem1)

                def body(gi, carry):
                    g = gi * 8
                    d = [pltpu.async_copy(p_rows.at[idx_s.at[g]], rows0,
                                          sem0),
                         pltpu.async_copy(p_rows.at[idx_s.at[g + 1]], rows1,
                                          sem1)]
                    for k in range(8):
                        d[k % 2].wait()
                        pltpu.sync_copy(bufs[k % 2],
                                        agg_sh.at[idx_d.at[g + k]], add=True)
                        if k + 2 < 8:
                            d[k % 2] = pltpu.async_copy(
                                p_rows.at[idx_s.at[g + k + 2]],
                                bufs[k % 2], sems[k % 2])
                    return carry

                lax.fori_loop(0, 5, body, 0)
            plsc.subcore_barrier()
            pltpu.sync_copy(
                agg_sh.at[pl.ds(s * TILE_ROWS, TILE_ROWS)],
                s_out.at[pl.ds(chunk * MPAD + s * TILE_ROWS, TILE_ROWS)])
            plsc.subcore_barrier()

    return _sc_degree, _sc_msgpass


def _tc_enc_body(x_ref, degp_ref, wn_ref, bn_ref, w0_ref, p_ref, dinv_ref):
    m = pl.program_id(0)
    deg = degp_ref[:, 0:1] + degp_ref[:, 1:2] + 1.0
    rows = jax.lax.broadcasted_iota(jnp.int32, (BM, 1), 0) + m * BM
    dinv = jnp.where(rows < N, lax.rsqrt(deg), 0.0)
    dinv_ref[...] = dinv
    h = jnp.maximum(
        jnp.dot(x_ref[...], wn_ref[...], preferred_element_type=jnp.float32)
        + bn_ref[...], 0.0)
    p = jnp.dot(h, w0_ref[...], preferred_element_type=jnp.float32) * dinv
    for cc in range(NCHUNK):
        p_ref[cc] = p[:, cc * 128:(cc + 1) * 128]


def _tc_combine_body(s_ref, p_ref, dinv_ref, b_ref, z_ref, stat_ref):
    m = pl.program_id(0)

    @pl.when(m == 0)
    def _():
        stat_ref[...] = jnp.zeros_like(stat_ref)

    dinv = dinv_ref[...]
    rows = jax.lax.broadcasted_iota(jnp.int32, (BM, 1), 0) + m * BM
    mask = rows < N
    for cc in range(NCHUNK):
        z = (s_ref[cc] + p_ref[cc]) * dinv + b_ref[:, cc * 128:(cc + 1) * 128]
        z_ref[:, cc * 128:(cc + 1) * 128] = z
        zm = jnp.where(mask, z, 0.0)
        stat_ref[0:1, cc * 128:(cc + 1) * 128] += jnp.sum(zm, axis=0,
                                                          keepdims=True)
        stat_ref[1:2, cc * 128:(cc + 1) * 128] += jnp.sum(zm * zm, axis=0,
                                                          keepdims=True)


def _bn_coeffs(stat, g, t):
    mu = stat[0:1, :] * (1.0 / N)
    var = stat[1:2, :] * (1.0 / N) - mu * mu
    a = lax.rsqrt(var + 1e-5) * g
    return a, t - mu * a


def _tc_normmm_body(z_ref, stat_ref, g_ref, t_ref, dinv_ref, w_ref, p_ref):
    a, c0 = _bn_coeffs(stat_ref[...], g_ref[...], t_ref[...])
    h = jnp.maximum(z_ref[...] * a + c0, 0.0)
    p = jnp.dot(h, w_ref[...], preferred_element_type=jnp.float32) \
        * dinv_ref[...]
    for cc in range(NCHUNK):
        p_ref[cc] = p[:, cc * 128:(cc + 1) * 128]


def _tc_final_body(z_ref, stat_ref, g_ref, t_ref, w_ref, b_ref, o_ref):
    a, c0 = _bn_coeffs(stat_ref[...], g_ref[...], t_ref[...])
    h = jnp.maximum(z_ref[...] * a + c0, 0.0)
    o_ref[...] = jnp.dot(h, w_ref[...],
                         preferred_element_type=jnp.float32) + b_ref[...]


def _row_spec(width):
    return pl.BlockSpec((BM, width), lambda m: (m, 0))


def _full_spec(shape):
    return pl.BlockSpec(shape, lambda m: tuple(0 for _ in shape))


_CHUNK_SPEC = pl.BlockSpec((NCHUNK, BM, 128), lambda m: (0, m, 0))
_P_SHAPE = jax.ShapeDtypeStruct((NCHUNK, MPAD, 128), jnp.float32)


def _tc_enc(x_pad, degp, w_node, b_node, w0):
    return pl.pallas_call(
        _tc_enc_body,
        grid=(GRID_M,),
        in_specs=[_row_spec(256), _row_spec(2), _full_spec((256, DH)),
                  _full_spec((1, DH)), _full_spec((DH, DH))],
        out_specs=[_CHUNK_SPEC, _row_spec(1)],
        out_shape=[_P_SHAPE, jax.ShapeDtypeStruct((MPAD, 1), jnp.float32)],
    )(x_pad, degp, w_node, b_node, w0)


def _tc_combine(s_agg, p, dinv, b):
    return pl.pallas_call(
        _tc_combine_body,
        grid=(GRID_M,),
        in_specs=[_CHUNK_SPEC, _CHUNK_SPEC, _row_spec(1), _full_spec((1, DH))],
        out_specs=[_row_spec(DH), _full_spec((8, DH))],
        out_shape=[jax.ShapeDtypeStruct((MPAD, DH), jnp.float32),
                   jax.ShapeDtypeStruct((8, DH), jnp.float32)],
    )(s_agg, p, dinv, b)


def _tc_normmm(z, stat, g, t, dinv, w):
    return pl.pallas_call(
        _tc_normmm_body,
        grid=(GRID_M,),
        in_specs=[_row_spec(DH), _full_spec((8, DH)), _full_spec((1, DH)),
                  _full_spec((1, DH)), _row_spec(1), _full_spec((DH, DH))],
        out_specs=_CHUNK_SPEC,
        out_shape=_P_SHAPE,
    )(z, stat, g, t, dinv, w)


def _tc_final(z, stat, g, t, w_cls, b_cls):
    return pl.pallas_call(
        _tc_final_body,
        grid=(GRID_M,),
        in_specs=[_row_spec(DH), _full_spec((8, DH)), _full_spec((1, DH)),
                  _full_spec((1, DH)), _full_spec((DH, 128)),
                  _full_spec((1, 128))],
        out_specs=_row_spec(128),
        out_shape=jax.ShapeDtypeStruct((MPAD, 128), jnp.float32),
    )(z, stat, g, t, w_cls, b_cls)


def kernel(x, edge_index, W_node, b_node, W0, b0, g0, t0, W1, b1, g1, t1,
           W2, b2, g2, t2, W_cls, b_cls):
    f32 = jnp.float32
    pad = (N + (jnp.arange(EPAD - E, dtype=jnp.int32) % NPADROWS))
    src = jnp.concatenate([edge_index[0], pad])
    dst = jnp.concatenate([edge_index[1], pad])
    dst_deg_idx = dst.reshape(32, 40, 128)
    chunk_off = (jnp.arange(NCHUNK, dtype=jnp.int32) * MPAD)[:, None]
    src_mp_idx = (src[None, :] + chunk_off).reshape(NCHUNK * 32, 40, 128)
    dst_mp_idx = dst.reshape(32, 40, 128)

    zeros1d = jnp.zeros((MPAD,), f32)
    ones1d = jnp.ones((128,), f32)
    zeros2d = jnp.zeros((TILE_ROWS, 128), f32)

    sc_degree, sc_msgpass = _sc_kernels()
    degp = sc_degree(dst_deg_idx, zeros1d, ones1d)
    degp_t = degp.T

    x_pad = jnp.pad(x, ((0, MPAD - N), (0, 0)))
    b_node2 = b_node.reshape(1, DH)

    p, dinv = _tc_enc(x_pad, degp_t, W_node, b_node2, W0)

    zs = None
    for (w_next, b_l, g_l, t_l) in ((W1, b0, g0, t0), (W2, b1, g1, t1),
                                    (None, b2, g2, t2)):
        s_flat = sc_msgpass(p.reshape(NCHUNK * MPAD, 128), src_mp_idx,
                            dst_mp_idx, zeros2d)
        s_agg = s_flat.reshape(NCHUNK, MPAD, 128)
        z, stat = _tc_combine(s_agg, p, dinv, b_l.reshape(1, DH))
        if w_next is not None:
            p = _tc_normmm(z, stat, g_l.reshape(1, DH), t_l.reshape(1, DH),
                           dinv, w_next)
        else:
            zs = (z, stat, g_l, t_l)

    z, stat, g_l, t_l = zs
    out = _tc_final(z, stat, g_l.reshape(1, DH), t_l.reshape(1, DH),
                    W_cls, b_cls.reshape(1, 128))
    return out[:N]


# speedup vs baseline: 11.9116x; 1.0428x over previous
import functools

import jax
import jax.numpy as jnp
from jax import lax
from jax.experimental import pallas as pl
from jax.experimental.pallas import tpu as pltpu
from jax.experimental.pallas import tpu_sc as plsc

N = 10000
MPAD = 10240
E = 160000
EPAD = 163840
NPADROWS = MPAD - N
DH = 512
NCHUNK = 4
BM = 256
GRID_M = MPAD // BM
TILE_ROWS = MPAD // 16


@functools.cache
def _sc_kernels():
    mesh = plsc.VectorSubcoreMesh(core_axis_name="c", subcore_axis_name="s",
                                  num_cores=2, num_subcores=16)

    @functools.partial(
        pl.kernel,
        out_type=jax.ShapeDtypeStruct((2, MPAD), jnp.float32),
        mesh=mesh,
        scratch_types=[
            pltpu.VMEM((40, 128), jnp.int32),
            pltpu.VMEM((128,), jnp.float32),
            pltpu.VMEM_SHARED((MPAD,), jnp.float32),
        ],
    )
    def _sc_degree(dst_idx, zeros1d, ones1d, deg_out, idx_v, ones_v, deg_sh):
        c = lax.axis_index("c")
        s = lax.axis_index("s")
        pltpu.sync_copy(zeros1d.at[pl.ds(0, TILE_ROWS)],
                        deg_sh.at[pl.ds(s * TILE_ROWS, TILE_ROWS)])
        pltpu.sync_copy(ones1d, ones_v)
        pltpu.sync_copy(dst_idx.at[c * 16 + s], idx_v)
        plsc.subcore_barrier()

        def body(j, carry):
            pltpu.sync_copy(ones_v, deg_sh.at[idx_v.at[j]], add=True)
            return carry

        lax.fori_loop(0, 40, body, 0)
        plsc.subcore_barrier()
        pltpu.sync_copy(deg_sh.at[pl.ds(s * TILE_ROWS, TILE_ROWS)],
                        deg_out.at[c, pl.ds(s * TILE_ROWS, TILE_ROWS)])

    @functools.partial(
        pl.kernel,
        out_type=jax.ShapeDtypeStruct((NCHUNK * MPAD, 128), jnp.float32),
        mesh=mesh,
        scratch_types=[
            pltpu.VMEM((40, 128), jnp.int32),
            pltpu.VMEM((40, 128), jnp.int32),
            pltpu.VMEM((128, 128), jnp.float32),
            pltpu.VMEM((128, 128), jnp.float32),
            pltpu.VMEM_SHARED((MPAD, 128), jnp.float32),
            pltpu.SemaphoreType.DMA,
            pltpu.SemaphoreType.DMA,
        ],
    )
    def _sc_msgpass(p_rows, src_idx, dst_idx, zeros2d, s_out,
                    idx_s, idx_d, rows0, rows1, agg_sh, sem0, sem1):
        c = lax.axis_index("c")
        s = lax.axis_index("s")
        for ci in range(2):
            chunk = c * 2 + ci
            pltpu.sync_copy(zeros2d,
                            agg_sh.at[pl.ds(s * TILE_ROWS, TILE_ROWS)])
            plsc.subcore_barrier()
            for half in range(2):
                pltpu.sync_copy(src_idx.at[(chunk * 16 + s) * 2 + half],
                                idx_s)
                pltpu.sync_copy(dst_idx.at[s * 2 + half], idx_d)

                bufs = (rows0, rows1)
                sems = (sem0, sem1)

                def body(gi, carry):
                    g = gi * 20
                    d = [pltpu.async_copy(p_rows.at[idx_s.at[g]], rows0,
                                          sem0),
                         pltpu.async_copy(p_rows.at[idx_s.at[g + 1]], rows1,
                                          sem1)]
                    for k in range(20):
                        d[k % 2].wait()
                        pltpu.sync_copy(bufs[k % 2],
                                        agg_sh.at[idx_d.at[g + k]], add=True)
                        if k + 2 < 20:
                            d[k % 2] = pltpu.async_copy(
                                p_rows.at[idx_s.at[g + k + 2]],
                                bufs[k % 2], sems[k % 2])
                    return carry

                lax.fori_loop(0, 2, body, 0)
            plsc.subcore_barrier()
            pltpu.sync_copy(
                agg_sh.at[pl.ds(s * TILE_ROWS, TILE_ROWS)],
                s_out.at[pl.ds(chunk * MPAD + s * TILE_ROWS, TILE_ROWS)])
            plsc.subcore_barrier()

    return _sc_degree, _sc_msgpass


def _tc_enc_body(x_ref, degp_ref, wn_ref, bn_ref, w0_ref, p_ref, dinv_ref):
    m = pl.program_id(0)
    deg = degp_ref[:, 0:1] + degp_ref[:, 1:2] + 1.0
    rows = jax.lax.broadcasted_iota(jnp.int32, (BM, 1), 0) + m * BM
    dinv = jnp.where(rows < N, lax.rsqrt(deg), 0.0)
    dinv_ref[...] = dinv
    bf16 = jnp.bfloat16
    h = jnp.maximum(
        jnp.dot(x_ref[...].astype(bf16), wn_ref[...].astype(bf16),
                preferred_element_type=jnp.float32) + bn_ref[...], 0.0)
    p = jnp.dot(h.astype(bf16), w0_ref[...].astype(bf16),
                preferred_element_type=jnp.float32) * dinv
    for cc in range(NCHUNK):
        p_ref[cc] = p[:, cc * 128:(cc + 1) * 128]


def _tc_combine_body(s_ref, p_ref, dinv_ref, b_ref, z_ref, stat_ref):
    m = pl.program_id(0)

    @pl.when(m == 0)
    def _():
        stat_ref[...] = jnp.zeros_like(stat_ref)

    dinv = dinv_ref[...]
    rows = jax.lax.broadcasted_iota(jnp.int32, (BM, 1), 0) + m * BM
    mask = rows < N
    for cc in range(NCHUNK):
        z = (s_ref[cc] + p_ref[cc]) * dinv + b_ref[:, cc * 128:(cc + 1) * 128]
        z_ref[:, cc * 128:(cc + 1) * 128] = z
        zm = jnp.where(mask, z, 0.0)
        stat_ref[0:1, cc * 128:(cc + 1) * 128] += jnp.sum(zm, axis=0,
                                                          keepdims=True)
        stat_ref[1:2, cc * 128:(cc + 1) * 128] += jnp.sum(zm * zm, axis=0,
                                                          keepdims=True)


def _bn_coeffs(stat, g, t):
    mu = stat[0:1, :] * (1.0 / N)
    var = stat[1:2, :] * (1.0 / N) - mu * mu
    a = lax.rsqrt(var + 1e-5) * g
    return a, t - mu * a


def _tc_normmm_body(z_ref, stat_ref, g_ref, t_ref, dinv_ref, w_ref, p_ref):
    a, c0 = _bn_coeffs(stat_ref[...], g_ref[...], t_ref[...])
    h = jnp.maximum(z_ref[...] * a + c0, 0.0)
    p = jnp.dot(h.astype(jnp.bfloat16), w_ref[...].astype(jnp.bfloat16),
                preferred_element_type=jnp.float32) * dinv_ref[...]
    for cc in range(NCHUNK):
        p_ref[cc] = p[:, cc * 128:(cc + 1) * 128]


def _tc_final_body(z_ref, stat_ref, g_ref, t_ref, w_ref, b_ref, o_ref):
    a, c0 = _bn_coeffs(stat_ref[...], g_ref[...], t_ref[...])
    h = jnp.maximum(z_ref[...] * a + c0, 0.0)
    o_ref[...] = jnp.dot(h.astype(jnp.bfloat16), w_ref[...].astype(jnp.bfloat16),
                         preferred_element_type=jnp.float32) + b_ref[...]


def _row_spec(width):
    return pl.BlockSpec((BM, width), lambda m: (m, 0))


def _full_spec(shape):
    return pl.BlockSpec(shape, lambda m: tuple(0 for _ in shape))


_CHUNK_SPEC = pl.BlockSpec((NCHUNK, BM, 128), lambda m: (0, m, 0))
_P_SHAPE = jax.ShapeDtypeStruct((NCHUNK, MPAD, 128), jnp.float32)


def _tc_enc(x_pad, degp, w_node, b_node, w0):
    return pl.pallas_call(
        _tc_enc_body,
        grid=(GRID_M,),
        in_specs=[_row_spec(256), _row_spec(2), _full_spec((256, DH)),
                  _full_spec((1, DH)), _full_spec((DH, DH))],
        out_specs=[_CHUNK_SPEC, _row_spec(1)],
        out_shape=[_P_SHAPE, jax.ShapeDtypeStruct((MPAD, 1), jnp.float32)],
    )(x_pad, degp, w_node, b_node, w0)


def _tc_combine(s_agg, p, dinv, b):
    return pl.pallas_call(
        _tc_combine_body,
        grid=(GRID_M,),
        in_specs=[_CHUNK_SPEC, _CHUNK_SPEC, _row_spec(1), _full_spec((1, DH))],
        out_specs=[_row_spec(DH), _full_spec((8, DH))],
        out_shape=[jax.ShapeDtypeStruct((MPAD, DH), jnp.float32),
                   jax.ShapeDtypeStruct((8, DH), jnp.float32)],
    )(s_agg, p, dinv, b)


def _tc_normmm(z, stat, g, t, dinv, w):
    return pl.pallas_call(
        _tc_normmm_body,
        grid=(GRID_M,),
        in_specs=[_row_spec(DH), _full_spec((8, DH)), _full_spec((1, DH)),
                  _full_spec((1, DH)), _row_spec(1), _full_spec((DH, DH))],
        out_specs=_CHUNK_SPEC,
        out_shape=_P_SHAPE,
    )(z, stat, g, t, dinv, w)


def _tc_final(z, stat, g, t, w_cls, b_cls):
    return pl.pallas_call(
        _tc_final_body,
        grid=(GRID_M,),
        in_specs=[_row_spec(DH), _full_spec((8, DH)), _full_spec((1, DH)),
                  _full_spec((1, DH)), _full_spec((DH, 128)),
                  _full_spec((1, 128))],
        out_specs=_row_spec(128),
        out_shape=jax.ShapeDtypeStruct((MPAD, 128), jnp.float32),
    )(z, stat, g, t, w_cls, b_cls)


def kernel(x, edge_index, W_node, b_node, W0, b0, g0, t0, W1, b1, g1, t1,
           W2, b2, g2, t2, W_cls, b_cls):
    f32 = jnp.float32
    pad = (N + (jnp.arange(EPAD - E, dtype=jnp.int32) % NPADROWS))
    src = jnp.concatenate([edge_index[0], pad])
    dst = jnp.concatenate([edge_index[1], pad])
    dst_deg_idx = dst.reshape(32, 40, 128)
    chunk_off = (jnp.arange(NCHUNK, dtype=jnp.int32) * MPAD)[:, None]
    src_mp_idx = (src[None, :] + chunk_off).reshape(NCHUNK * 32, 40, 128)
    dst_mp_idx = dst.reshape(32, 40, 128)

    zeros1d = jnp.zeros((MPAD,), f32)
    ones1d = jnp.ones((128,), f32)
    zeros2d = jnp.zeros((TILE_ROWS, 128), f32)

    sc_degree, sc_msgpass = _sc_kernels()
    degp = sc_degree(dst_deg_idx, zeros1d, ones1d)
    degp_t = degp.T

    x_pad = jnp.pad(x, ((0, MPAD - N), (0, 0)))
    b_node2 = b_node.reshape(1, DH)

    p, dinv = _tc_enc(x_pad, degp_t, W_node, b_node2, W0)

    zs = None
    for (w_next, b_l, g_l, t_l) in ((W1, b0, g0, t0), (W2, b1, g1, t1),
                                    (None, b2, g2, t2)):
        s_flat = sc_msgpass(p.reshape(NCHUNK * MPAD, 128), src_mp_idx,
                            dst_mp_idx, zeros2d)
        s_agg = s_flat.reshape(NCHUNK, MPAD, 128)
        z, stat = _tc_combine(s_agg, p, dinv, b_l.reshape(1, DH))
        if w_next is not None:
            p = _tc_normmm(z, stat, g_l.reshape(1, DH), t_l.reshape(1, DH),
                           dinv, w_next)
        else:
            zs = (z, stat, g_l, t_l)

    z, stat, g_l, t_l = zs
    out = _tc_final(z, stat, g_l.reshape(1, DH), t_l.reshape(1, DH),
                    W_cls, b_cls.reshape(1, 128))
    return out[:N]
